# Initial kernel scaffold; baseline (speedup 1.0000x reference)
import jax
import jax.numpy as jnp
from jax.experimental import pallas as pl


def kernel(inputs, linear_W, linear_b, w0_W, w0_b, w_bn_g, w_bn_b, w3_W, w3_b, norm_g, norm_b, fc1_W, fc1_b, fc2_W, fc2_b, unq_inv):
    raise NotImplementedError("write your pallas kernel here")


# speedup vs baseline: 1.0919x; 1.0919x over previous
import jax
import jax.numpy as jnp
from jax.experimental import pallas as pl

N = 800000
S = 20000
IN_C = 10
C = 32
BLK = 8000


def _dense_kernel(x_ref, WlT_ref, bl_ref, a1_ref, c1_ref,
                  W0T_ref, b0_ref, a2_ref, c2_ref, w3T_ref, b3_ref,
                  nv_ref, v_ref, wgt_ref):
    x = x_ref[...]
    v = jnp.dot(x, WlT_ref[...], preferred_element_type=jnp.float32) + bl_ref[...]
    v_ref[...] = v
    nv_ref[...] = v * a1_ref[...] + c1_ref[...]
    u = jnp.dot(x, W0T_ref[...], preferred_element_type=jnp.float32) + b0_ref[...]
    h = jnp.maximum(u * a2_ref[...] + c2_ref[...], 0.0)
    wgt_ref[...] = jnp.dot(h, w3T_ref[...], preferred_element_type=jnp.float32) + b3_ref[...]


def kernel(inputs, linear_W, linear_b, w0_W, w0_b, w_bn_g, w_bn_b, w3_W, w3_b,
           norm_g, norm_b, fc1_W, fc1_b, fc2_W, fc2_b, unq_inv):
    seg = unq_inv.astype(jnp.int32)
    n = inputs.shape[0]

    m = jnp.mean(inputs, axis=0)
    M2 = (inputs.T @ inputs) / n

    def bn_affine(W, b, g, beta, eps):
        Wm = W @ m
        mean_v = Wm + b
        e2 = jnp.einsum('ci,ij,cj->c', W, M2, W) + 2.0 * b * Wm + b * b
        var = jnp.maximum(e2 - mean_v * mean_v, 0.0)
        a = g / jnp.sqrt(var + eps)
        c = beta - mean_v * a
        return a, c

    a1, c1 = bn_affine(linear_W, linear_b, norm_g, norm_b, 1e-3)
    a2, c2 = bn_affine(w0_W, w0_b, w_bn_g, w_bn_b, 1e-5)

    grid = (n // BLK,)
    row = lambda i: (i, 0)
    fix = lambda i: (0, 0)
    nv, v, wgt = pl.pallas_call(
        _dense_kernel,
        grid=grid,
        in_specs=[
            pl.BlockSpec((BLK, IN_C), row),
            pl.BlockSpec((IN_C, C), fix),
            pl.BlockSpec((1, C), fix),
            pl.BlockSpec((1, C), fix),
            pl.BlockSpec((1, C), fix),
            pl.BlockSpec((IN_C, C), fix),
            pl.BlockSpec((1, C), fix),
            pl.BlockSpec((1, C), fix),
            pl.BlockSpec((1, C), fix),
            pl.BlockSpec((C, 1), fix),
            pl.BlockSpec((1, 1), fix),
        ],
        out_specs=[
            pl.BlockSpec((BLK, C), row),
            pl.BlockSpec((BLK, C), row),
            pl.BlockSpec((BLK, 1), row),
        ],
        out_shape=[
            jax.ShapeDtypeStruct((n, C), jnp.float32),
            jax.ShapeDtypeStruct((n, C), jnp.float32),
            jax.ShapeDtypeStruct((n, 1), jnp.float32),
        ],
    )(
        inputs,
        linear_W.T, linear_b[None, :], a1[None, :], c1[None, :],
        w0_W.T, w0_b[None, :], a2[None, :], c2[None, :],
        w3_W.T, w3_b[None, :],
    )

    x = jnp.maximum(nv, 0.0)

    counts = jax.ops.segment_sum(jnp.ones((n, 1), jnp.float32), seg,
                                 num_segments=S, indices_are_sorted=True)
    nonempty = counts > 0

    q_max = jax.ops.segment_max(x, seg, num_segments=S, indices_are_sorted=True)
    q_max = jnp.where(nonempty, q_max, 0.0)

    seg_max_w = jax.ops.segment_max(wgt, seg, num_segments=S, indices_are_sorted=True)
    seg_max_w = jnp.where(nonempty, seg_max_w, 0.0)
    ex = jnp.exp(wgt - seg_max_w[seg])
    denom = jax.ops.segment_sum(ex, seg, num_segments=S, indices_are_sorted=True)
    soft_weight = ex / jnp.maximum(denom, 1e-12)[seg]

    seg_sum = jax.ops.segment_sum(soft_weight * v, seg, num_segments=S,
                                  indices_are_sorted=True)
    weight_x = seg_sum / jnp.maximum(counts, 1.0)

    min_feat = -jax.ops.segment_max(-nv, seg, num_segments=S, indices_are_sorted=True)
    min_feat = jnp.where(nonempty, min_feat, 0.0)
    cw = jax.nn.sigmoid(jax.nn.relu(min_feat @ fc1_W.T + fc1_b) @ fc2_W.T + fc2_b)

    final_feat = cw * q_max + (1.0 - cw) * weight_x
    return jnp.concatenate([x, final_feat[seg]], axis=-1)
